# baseline (speedup 1.0000x reference)
import jax
import jax.numpy as jnp
import numpy as np
from jax import lax
from jax.experimental import pallas as pl
from jax.experimental.pallas import tpu as pltpu

_KS0 = np.int32(0)
_KS1 = np.int32(42)
_KS2 = np.int32(np.uint32(0x1BD11BDA ^ 42).view(np.int32))
_ROTS_A = (13, 15, 26, 6)
_ROTS_B = (17, 29, 16, 24)
_THRESH = np.int32(419431)

_XCH = 10


def _keep_bits(j):
    x0 = jnp.zeros_like(j)
    x1 = j + _KS1
    inj = ((_KS1, _KS2, 1), (_KS2, _KS0, 2), (_KS0, _KS1, 3),
           (_KS1, _KS2, 4), (_KS2, _KS0, 5))
    for i, (ka, kb, cnt) in enumerate(inj):
        for r in (_ROTS_A if i % 2 == 0 else _ROTS_B):
            x0 = x0 + x1
            x1 = (x1 << r) | lax.shift_right_logical(x1, 32 - r)
            x1 = x1 ^ x0
        x0 = x0 + ka
        x1 = x1 + jnp.int32(kb + np.int32(cnt))
    return lax.shift_right_logical(x0 ^ x1, 9) >= _THRESH


def _body(x_in, e_in, y_in, tr_in, te_in,
          x_out, e_out, y_out, tr_out, te_out,
          xbuf, ebuf, ybuf,
          sx_in, sx_out, se_in, se_out, sy_in, sy_out):
    n = x_in.shape[0]
    rows = n // _XCH

    x_ins, x_outs = [], []
    for k in range(_XCH):
        sl = pl.ds(k * rows, rows)
        x_ins.append(pltpu.make_async_copy(
            x_in.at[sl], xbuf.at[sl], sx_in.at[k]))
        x_outs.append(pltpu.make_async_copy(
            xbuf.at[sl], x_out.at[sl], sx_out.at[k]))
    e_cin = pltpu.make_async_copy(e_in, ebuf, se_in)
    e_cout = pltpu.make_async_copy(ebuf, e_out, se_out)
    y_cin = pltpu.make_async_copy(y_in, ybuf, sy_in)
    y_cout = pltpu.make_async_copy(ybuf, y_out, sy_out)

    for cp in x_ins:
        cp.start()
    e_cin.start()
    y_cin.start()

    keep = _keep_bits(lax.broadcasted_iota(jnp.int32, (tr_in.shape[0],), 0))
    tr32 = tr_in[...].astype(jnp.int32)
    te32 = te_in[...].astype(jnp.int32)
    tr_out[...] = jnp.where(keep, tr32, 0).astype(jnp.int8)
    te_out[...] = jnp.where(keep, te32, 0).astype(jnp.int8)

    for k in range(_XCH):
        x_ins[k].wait()
        x_outs[k].start()
    e_cin.wait()
    e_cout.start()
    y_cin.wait()
    y_cout.start()

    for cp in x_outs:
        cp.wait()
    e_cout.wait()
    y_cout.wait()


def _make(n, d, e):
    any_spec = pl.BlockSpec(memory_space=pl.ANY)
    vmem_spec = pl.BlockSpec(memory_space=pltpu.MemorySpace.VMEM)
    return pl.pallas_call(
        _body,
        in_specs=[any_spec, any_spec, any_spec, vmem_spec, vmem_spec],
        out_specs=[any_spec, any_spec, any_spec, vmem_spec, vmem_spec],
        out_shape=[
            jax.ShapeDtypeStruct((n, d), jnp.float32),
            jax.ShapeDtypeStruct((2, e), jnp.int32),
            jax.ShapeDtypeStruct((n,), jnp.int32),
            jax.ShapeDtypeStruct((n,), jnp.int8),
            jax.ShapeDtypeStruct((n,), jnp.int8),
        ],
        scratch_shapes=[
            pltpu.VMEM((n, d), jnp.float32),
            pltpu.VMEM((2, e), jnp.int32),
            pltpu.VMEM((n,), jnp.int32),
            pltpu.SemaphoreType.DMA((_XCH,)),
            pltpu.SemaphoreType.DMA((_XCH,)),
            pltpu.SemaphoreType.DMA,
            pltpu.SemaphoreType.DMA,
            pltpu.SemaphoreType.DMA,
            pltpu.SemaphoreType.DMA,
        ],
    )


def kernel(x, y, train_mask, test_mask, edge_index):
    n, d = x.shape
    e = edge_index.shape[1]
    x_o, e_o, y_o, tr_o, te_o = _make(n, d, e)(
        x, edge_index, y,
        train_mask.view(jnp.int8), test_mask.view(jnp.int8))
    return (x_o, e_o, y_o, tr_o.view(jnp.bool_), te_o.view(jnp.bool_))


# speedup vs baseline: 1.0250x; 1.0250x over previous
import jax
import jax.numpy as jnp
import numpy as np
from jax import lax
from jax.experimental import pallas as pl
from jax.experimental.pallas import tpu as pltpu

_KS0 = np.int32(0)
_KS1 = np.int32(42)
_KS2 = np.int32(np.uint32(0x1BD11BDA ^ 42).view(np.int32))
_ROTS_A = (13, 15, 26, 6)
_ROTS_B = (17, 29, 16, 24)
_THRESH = np.int32(419431)

_XCH = 5


def _keep_bits(j):
    x0 = jnp.zeros_like(j)
    x1 = j + _KS1
    inj = ((_KS1, _KS2, 1), (_KS2, _KS0, 2), (_KS0, _KS1, 3),
           (_KS1, _KS2, 4), (_KS2, _KS0, 5))
    for i, (ka, kb, cnt) in enumerate(inj):
        for r in (_ROTS_A if i % 2 == 0 else _ROTS_B):
            x0 = x0 + x1
            x1 = (x1 << r) | lax.shift_right_logical(x1, 32 - r)
            x1 = x1 ^ x0
        x0 = x0 + ka
        x1 = x1 + jnp.int32(kb + np.int32(cnt))
    return lax.shift_right_logical(x0 ^ x1, 9) >= _THRESH


def _body(x_in, e_in, y_in, tr_in, te_in,
          x_out, e_out, y_out, tr_out, te_out,
          xbuf, ebuf, ybuf,
          sx_in, sx_out, se_in, se_out, sy_in, sy_out):
    n = x_in.shape[0]
    rows = n // _XCH

    x_ins, x_outs = [], []
    for k in range(_XCH):
        sl = pl.ds(k * rows, rows)
        x_ins.append(pltpu.make_async_copy(
            x_in.at[sl], xbuf.at[sl], sx_in.at[k]))
        x_outs.append(pltpu.make_async_copy(
            xbuf.at[sl], x_out.at[sl], sx_out.at[k]))
    e_cin = pltpu.make_async_copy(e_in, ebuf, se_in)
    e_cout = pltpu.make_async_copy(ebuf, e_out, se_out)
    y_cin = pltpu.make_async_copy(y_in, ybuf, sy_in)
    y_cout = pltpu.make_async_copy(ybuf, y_out, sy_out)

    for cp in x_ins:
        cp.start()
    e_cin.start()
    y_cin.start()

    keep = _keep_bits(lax.broadcasted_iota(jnp.int32, (tr_in.shape[0],), 0))
    tr32 = tr_in[...].astype(jnp.int32)
    te32 = te_in[...].astype(jnp.int32)
    tr_out[...] = jnp.where(keep, tr32, 0).astype(jnp.int8)
    te_out[...] = jnp.where(keep, te32, 0).astype(jnp.int8)

    for k in range(_XCH):
        x_ins[k].wait()
        x_outs[k].start()
    e_cin.wait()
    e_cout.start()
    y_cin.wait()
    y_cout.start()

    for cp in x_outs:
        cp.wait()
    e_cout.wait()
    y_cout.wait()


def _make(n, d, e):
    any_spec = pl.BlockSpec(memory_space=pl.ANY)
    vmem_spec = pl.BlockSpec(memory_space=pltpu.MemorySpace.VMEM)
    return pl.pallas_call(
        _body,
        in_specs=[any_spec, any_spec, any_spec, vmem_spec, vmem_spec],
        out_specs=[any_spec, any_spec, any_spec, vmem_spec, vmem_spec],
        out_shape=[
            jax.ShapeDtypeStruct((n, d), jnp.float32),
            jax.ShapeDtypeStruct((2, e), jnp.int32),
            jax.ShapeDtypeStruct((n,), jnp.int32),
            jax.ShapeDtypeStruct((n,), jnp.int8),
            jax.ShapeDtypeStruct((n,), jnp.int8),
        ],
        scratch_shapes=[
            pltpu.VMEM((n, d), jnp.float32),
            pltpu.VMEM((2, e), jnp.int32),
            pltpu.VMEM((n,), jnp.int32),
            pltpu.SemaphoreType.DMA((_XCH,)),
            pltpu.SemaphoreType.DMA((_XCH,)),
            pltpu.SemaphoreType.DMA,
            pltpu.SemaphoreType.DMA,
            pltpu.SemaphoreType.DMA,
            pltpu.SemaphoreType.DMA,
        ],
    )


def kernel(x, y, train_mask, test_mask, edge_index):
    n, d = x.shape
    e = edge_index.shape[1]
    x_o, e_o, y_o, tr_o, te_o = _make(n, d, e)(
        x, edge_index, y,
        train_mask.view(jnp.int8), test_mask.view(jnp.int8))
    return (x_o, e_o, y_o, tr_o.view(jnp.bool_), te_o.view(jnp.bool_))
